# baseline (speedup 1.0000x reference)
import functools

import jax
import jax.numpy as jnp
from jax import lax
from jax.experimental import pallas as pl
from jax.experimental.pallas import tpu as pltpu
from jax.experimental.pallas import tpu_sc as plsc

N = 10000
C = 128
NC = 2
NS = 16
ZR = 16
RPT = 624
TAIL = N - NS * RPT
K = 80


def _mm_pm_body(x_ref, w_ref, op_ref, on_ref):
    a = jnp.dot(x_ref[...], w_ref[...], preferred_element_type=jnp.float32)
    op_ref[...] = a
    on_ref[...] = -a


def _sig_mm_pm_body(x_ref, w_ref, op_ref, on_ref):
    s = jax.nn.sigmoid(x_ref[...])
    a = jnp.dot(s, w_ref[...], preferred_element_type=jnp.float32)
    op_ref[...] = a
    on_ref[...] = -a


def _mm_sig_body(x_ref, w_ref, o_ref):
    o_ref[...] = jax.nn.sigmoid(
        jnp.dot(x_ref[...], w_ref[...], preferred_element_type=jnp.float32))


def _mm_add_sig_body(x_ref, w_ref, b_ref, o_ref):
    o_ref[...] = jax.nn.sigmoid(
        jnp.dot(x_ref[...], w_ref[...], preferred_element_type=jnp.float32)
        + b_ref[...])


def _row_spec(blk):
    return pl.BlockSpec((blk, C), lambda i: (i, 0))


def _w_spec():
    return pl.BlockSpec((C, C), lambda i: (0, 0))


def _tc_pm(body, x, w, blk):
    rows = x.shape[0]
    return pl.pallas_call(
        body,
        grid=(rows // blk,),
        in_specs=[_row_spec(blk), _w_spec()],
        out_specs=[_row_spec(blk), _row_spec(blk)],
        out_shape=[jax.ShapeDtypeStruct((rows, C), jnp.float32)] * 2,
    )(x, w)


def _tc_mm_sig(x, w, blk):
    rows = x.shape[0]
    return pl.pallas_call(
        _mm_sig_body,
        grid=(rows // blk,),
        in_specs=[_row_spec(blk), _w_spec()],
        out_specs=_row_spec(blk),
        out_shape=jax.ShapeDtypeStruct((rows, C), jnp.float32),
    )(x, w)


def _tc_mm_add_sig(x, w, b, blk):
    rows = x.shape[0]
    return pl.pallas_call(
        _mm_add_sig_body,
        grid=(rows // blk,),
        in_specs=[_row_spec(blk), _w_spec(), _row_spec(blk)],
        out_specs=_row_spec(blk),
        out_shape=jax.ShapeDtypeStruct((rows, C), jnp.float32),
    )(x, w, b)


def _zero_acc_slice(acc, zbuf, tid, sem):
    def zb(i, _):
        zbuf[i // (C // 16), pl.ds((i % (C // 16)) * 16, 16)] = (
            jnp.zeros((16,), jnp.float32))
        return 0

    lax.fori_loop(0, ZR * (C // 16), zb, 0)

    def zcopy(i, _):
        pltpu.async_copy(zbuf, acc.at[pl.ds(tid * RPT + i * ZR, ZR)], sem)
        return 0

    lax.fori_loop(0, RPT // ZR, zcopy, 0)

    def zdrain(i, _):
        pltpu.make_async_copy(zbuf, acc.at[pl.ds(tid * RPT + i * ZR, ZR)],
                              sem).wait()
        return 0

    lax.fori_loop(0, RPT // ZR, zdrain, 0)

    @pl.when(tid == NS - 1)
    def _():
        pltpu.sync_copy(zbuf.at[pl.ds(0, TAIL)], acc.at[pl.ds(NS * RPT, TAIL)])


def _acc_writeback(acc, out_hbm, tid):
    pltpu.sync_copy(acc.at[pl.ds(tid * RPT, RPT)],
                    out_hbm.at[pl.ds(tid * RPT, RPT)])

    @pl.when(tid == NS - 1)
    def _():
        pltpu.sync_copy(acc.at[pl.ds(NS * RPT, TAIL)],
                        out_hbm.at[pl.ds(NS * RPT, TAIL)])


def _ring_scatter(read_src_fn, didx_hbm, acc, rows0, rows1, didx0, didx1,
                  semg0, semg1, sems0, sems1, nch, base_t):
    def issue_read(i, rows, semg):
        pltpu.async_copy(read_src_fn(i), rows, semg)

    def wait_read(i, rows, semg):
        pltpu.make_async_copy(read_src_fn(i), rows, semg).wait()

    def issue_scat(rows, didx, sems):
        pltpu.async_copy(rows, acc.at[didx], sems, add=True)

    def wait_scat(rows, didx, sems):
        pltpu.make_async_copy(rows, acc.at[didx], sems).wait()

    pltpu.sync_copy(didx_hbm.at[pl.ds(base_t, K)], didx0)
    issue_read(0, rows0, semg0)

    def body(g, _):
        i0 = 2 * g
        i1 = i0 + 1
        wait_read(i0, rows0, semg0)
        issue_scat(rows0, didx0, sems0)

        @pl.when(g > 0)
        def _():
            wait_scat(rows1, didx1, sems1)

        pltpu.sync_copy(didx_hbm.at[pl.ds(base_t + i1 * K, K)], didx1)
        issue_read(i1, rows1, semg1)
        wait_read(i1, rows1, semg1)
        issue_scat(rows1, didx1, sems1)
        wait_scat(rows0, didx0, sems0)

        @pl.when(i1 + 1 < nch)
        def _():
            pltpu.sync_copy(didx_hbm.at[pl.ds(base_t + (i1 + 1) * K, K)],
                            didx0)
            issue_read(i1 + 1, rows0, semg0)

        return 0

    lax.fori_loop(0, nch // 2, body, 0)
    if nch % 2 == 1:
        i = nch - 1
        wait_read(i, rows0, semg0)
        issue_scat(rows0, didx0, sems0)
        wait_scat(rows1, didx1, sems1)
        wait_scat(rows0, didx0, sems0)
    else:
        wait_scat(rows1, didx1, sems1)


def _gather_diff_core(pos_hbm, neg_hbm, iv_hbm, iu_hbm, out_hbm, iall,
                      rows0, rows1, semg0, semg1, sema0, sema1, semw0, semw1,
                      tid, ept):
    nch = ept // K
    base_t = tid * ept
    pltpu.sync_copy(iv_hbm.at[pl.ds(base_t, ept)], iall.at[pl.ds(0, ept)])
    pltpu.sync_copy(iu_hbm.at[pl.ds(base_t, ept)], iall.at[pl.ds(ept, ept)])

    def g1_src(i):
        return pos_hbm.at[iall.at[pl.ds(i * K, K)]]

    def g2_src(i):
        return neg_hbm.at[iall.at[pl.ds(ept + i * K, K)]]

    def out_dst(i):
        return out_hbm.at[pl.ds(base_t + i * K, K)]

    pltpu.async_copy(g1_src(0), rows0, semg0)

    def body(g, _):
        i0 = 2 * g
        i1 = i0 + 1
        pltpu.make_async_copy(g1_src(i0), rows0, semg0).wait()
        pltpu.async_copy(g2_src(i0), rows0, sema0, add=True)

        @pl.when(g > 0)
        def _():
            pltpu.make_async_copy(rows1, out_dst(i1 - 2), semw1).wait()

        pltpu.async_copy(g1_src(i1), rows1, semg1)
        pltpu.make_async_copy(g2_src(i0), rows0, sema0).wait()
        pltpu.async_copy(rows0, out_dst(i0), semw0)
        pltpu.make_async_copy(g1_src(i1), rows1, semg1).wait()
        pltpu.async_copy(g2_src(i1), rows1, sema1, add=True)

        @pl.when(i1 + 1 < nch)
        def _():
            pltpu.make_async_copy(rows0, out_dst(i0), semw0).wait()
            pltpu.async_copy(g1_src(i1 + 1), rows0, semg0)

        pltpu.make_async_copy(g2_src(i1), rows1, sema1).wait()
        pltpu.async_copy(rows1, out_dst(i1), semw1)
        return 0

    lax.fori_loop(0, nch // 2, body, 0)
    i = nch - 1
    pltpu.make_async_copy(g1_src(i), rows0, semg0).wait()
    pltpu.async_copy(g2_src(i), rows0, sema0, add=True)
    pltpu.make_async_copy(g2_src(i), rows0, sema0).wait()
    pltpu.async_copy(rows0, out_dst(i), semw0)
    pltpu.make_async_copy(rows0, out_dst(i), semw0).wait()
    pltpu.make_async_copy(rows1, out_dst(i - 1), semw1).wait()


def _sc_stage_a(x, xw2, xw2n, adj_src, adj_dst, inc_v, inc_u):
    EA = adj_src.shape[0]
    EI = inc_v.shape[0]
    ept_a = EA // NS
    ept_i = EI // NS
    mesh = plsc.VectorSubcoreMesh(core_axis_name="c", subcore_axis_name="s")

    @functools.partial(
        pl.kernel,
        out_type=[jax.ShapeDtypeStruct((N, C), jnp.float32),
                  jax.ShapeDtypeStruct((EI, C), jnp.float32)],
        mesh=mesh,
        scratch_types=[
            pltpu.VMEM_SHARED((N, C), jnp.float32),
            pltpu.VMEM((2 * ept_i,), jnp.int32),
            pltpu.VMEM((K, C), jnp.float32),
            pltpu.VMEM((K, C), jnp.float32),
            pltpu.VMEM((K,), jnp.int32),
            pltpu.VMEM((K,), jnp.int32),
            pltpu.VMEM((ZR, C), jnp.float32),
            [pltpu.SemaphoreType.DMA] * 7,
        ],
    )
    def k(x_hbm, xw2_hbm, xw2n_hbm, asrc_hbm, adst_hbm, iv_hbm, iu_hbm,
          gx_hbm, epre_hbm, acc, iall, rows0, rows1, didx0, didx1, zbuf,
          sems):
        cid = lax.axis_index("c")
        tid = lax.axis_index("s")

        @pl.when(cid == 0)
        def _():
            nch = ept_a // K
            base_t = tid * ept_a
            _zero_acc_slice(acc, zbuf, tid, sems[6])
            pltpu.sync_copy(asrc_hbm.at[pl.ds(base_t, ept_a)],
                            iall.at[pl.ds(0, ept_a)])
            plsc.subcore_barrier()

            def read_src(i):
                return x_hbm.at[iall.at[pl.ds(i * K, K)]]

            _ring_scatter(read_src, adst_hbm, acc, rows0, rows1, didx0,
                          didx1, sems[0], sems[1], sems[2], sems[3], nch,
                          base_t)
            plsc.subcore_barrier()
            _acc_writeback(acc, gx_hbm, tid)

        @pl.when(cid == 1)
        def _():
            _gather_diff_core(xw2_hbm, xw2n_hbm, iv_hbm, iu_hbm, epre_hbm,
                              iall, rows0, rows1, sems[0], sems[1], sems[2],
                              sems[3], sems[4], sems[5], tid, ept_i)

    return k(x, xw2, xw2n, adj_src, adj_dst, inc_v, inc_u)


def _sc_stage_c(n1, ewp, ewn, adj_src, adj_dst, inc_v, inc_u):
    EA = adj_src.shape[0]
    EI = inc_v.shape[0]
    ept_a = EA // NS
    ept_i = EI // NS
    mesh = plsc.VectorSubcoreMesh(core_axis_name="c", subcore_axis_name="s")

    @functools.partial(
        pl.kernel,
        out_type=[jax.ShapeDtypeStruct((N, C), jnp.float32),
                  jax.ShapeDtypeStruct((N, C), jnp.float32)],
        mesh=mesh,
        scratch_types=[
            pltpu.VMEM_SHARED((N, C), jnp.float32),
            pltpu.VMEM((ept_a,), jnp.int32),
            pltpu.VMEM((K, C), jnp.float32),
            pltpu.VMEM((K, C), jnp.float32),
            pltpu.VMEM((K,), jnp.int32),
            pltpu.VMEM((K,), jnp.int32),
            pltpu.VMEM((ZR, C), jnp.float32),
            [pltpu.SemaphoreType.DMA] * 7,
        ],
    )
    def k(n1_hbm, ewp_hbm, ewn_hbm, asrc_hbm, adst_hbm, iv_hbm, iu_hbm,
          gn_hbm, e2_hbm, acc, iall, rows0, rows1, didx0, didx1, zbuf, sems):
        cid = lax.axis_index("c")
        tid = lax.axis_index("s")

        @pl.when(cid == 0)
        def _():
            nch = ept_a // K
            base_t = tid * ept_a
            _zero_acc_slice(acc, zbuf, tid, sems[6])
            pltpu.sync_copy(asrc_hbm.at[pl.ds(base_t, ept_a)],
                            iall.at[pl.ds(0, ept_a)])
            plsc.subcore_barrier()

            def read_src(i):
                return n1_hbm.at[iall.at[pl.ds(i * K, K)]]

            _ring_scatter(read_src, adst_hbm, acc, rows0, rows1, didx0,
                          didx1, sems[0], sems[1], sems[2], sems[3], nch,
                          base_t)
            plsc.subcore_barrier()
            _acc_writeback(acc, gn_hbm, tid)

        @pl.when(cid == 1)
        def _():
            nch = ept_i // K
            base_t = tid * ept_i
            _zero_acc_slice(acc, zbuf, tid, sems[6])
            plsc.subcore_barrier()

            def read_p(i):
                return ewp_hbm.at[pl.ds(base_t + i * K, K)]

            def read_n(i):
                return ewn_hbm.at[pl.ds(base_t + i * K, K)]

            _ring_scatter(read_p, iv_hbm, acc, rows0, rows1, didx0, didx1,
                          sems[0], sems[1], sems[2], sems[3], nch, base_t)
            _ring_scatter(read_n, iu_hbm, acc, rows0, rows1, didx0, didx1,
                          sems[0], sems[1], sems[2], sems[3], nch, base_t)
            plsc.subcore_barrier()
            _acc_writeback(acc, e2_hbm, tid)

    return k(n1, ewp, ewn, adj_src, adj_dst, inc_v, inc_u)


def kernel(x, adj_src, adj_dst, inc_u, inc_v, W1, W2, W3, W4):
    xw2, xw2n = _tc_pm(_mm_pm_body, x, W2, blk=1000)
    gx, e_pre = _sc_stage_a(x, xw2, xw2n, adj_src, adj_dst, inc_v, inc_u)
    n1 = _tc_mm_sig(gx, W1, blk=1000)
    ewp, ewn = _tc_pm(_sig_mm_pm_body, e_pre, W4, blk=2000)
    gn, e2 = _sc_stage_c(n1, ewp, ewn, adj_src, adj_dst, inc_v, inc_u)
    return _tc_mm_add_sig(gn, W3, e2, blk=1000)


# speedup vs baseline: 1.6097x; 1.6097x over previous
import functools

import jax
import jax.numpy as jnp
from jax import lax
from jax.experimental import pallas as pl
from jax.experimental.pallas import tpu as pltpu
from jax.experimental.pallas import tpu_sc as plsc

N = 10000
C = 128
NC = 2
NS = 16
ZR = 16
RPT = 624
TAIL = N - NS * RPT
K = 80


def _mm_pm_body(x_ref, w_ref, op_ref, on_ref):
    a = jnp.dot(x_ref[...], w_ref[...], preferred_element_type=jnp.float32)
    op_ref[...] = a
    on_ref[...] = -a


def _sig_mm_pm_body(x_ref, w_ref, op_ref, on_ref):
    s = jax.nn.sigmoid(x_ref[...])
    a = jnp.dot(s, w_ref[...], preferred_element_type=jnp.float32)
    op_ref[...] = a
    on_ref[...] = -a


def _mm_sig_body(x_ref, w_ref, o_ref):
    o_ref[...] = jax.nn.sigmoid(
        jnp.dot(x_ref[...], w_ref[...], preferred_element_type=jnp.float32))


def _mm_add_sig_body(x_ref, w_ref, b_ref, o_ref):
    o_ref[...] = jax.nn.sigmoid(
        jnp.dot(x_ref[...], w_ref[...], preferred_element_type=jnp.float32)
        + b_ref[...])


def _row_spec(blk):
    return pl.BlockSpec((blk, C), lambda i: (i, 0))


def _w_spec():
    return pl.BlockSpec((C, C), lambda i: (0, 0))


def _tc_pm(body, x, w, blk):
    rows = x.shape[0]
    return pl.pallas_call(
        body,
        grid=(rows // blk,),
        in_specs=[_row_spec(blk), _w_spec()],
        out_specs=[_row_spec(blk), _row_spec(blk)],
        out_shape=[jax.ShapeDtypeStruct((rows, C), jnp.float32)] * 2,
    )(x, w)


def _tc_mm_sig(x, w, blk):
    rows = x.shape[0]
    return pl.pallas_call(
        _mm_sig_body,
        grid=(rows // blk,),
        in_specs=[_row_spec(blk), _w_spec()],
        out_specs=_row_spec(blk),
        out_shape=jax.ShapeDtypeStruct((rows, C), jnp.float32),
    )(x, w)


def _tc_mm_add_sig(x, w, b, blk):
    rows = x.shape[0]
    return pl.pallas_call(
        _mm_add_sig_body,
        grid=(rows // blk,),
        in_specs=[_row_spec(blk), _w_spec(), _row_spec(blk)],
        out_specs=_row_spec(blk),
        out_shape=jax.ShapeDtypeStruct((rows, C), jnp.float32),
    )(x, w, b)


def _zero_acc_slice(acc, zbuf, tid, sem):
    def zb(i, _):
        zbuf[i // (C // 16), pl.ds((i % (C // 16)) * 16, 16)] = (
            jnp.zeros((16,), jnp.float32))
        return 0

    lax.fori_loop(0, ZR * (C // 16), zb, 0)

    def zcopy(i, _):
        pltpu.async_copy(zbuf, acc.at[pl.ds(tid * RPT + i * ZR, ZR)], sem)
        return 0

    lax.fori_loop(0, RPT // ZR, zcopy, 0)

    def zdrain(i, _):
        pltpu.make_async_copy(zbuf, acc.at[pl.ds(tid * RPT + i * ZR, ZR)],
                              sem).wait()
        return 0

    lax.fori_loop(0, RPT // ZR, zdrain, 0)

    @pl.when(tid == NS - 1)
    def _():
        pltpu.sync_copy(zbuf.at[pl.ds(0, TAIL)], acc.at[pl.ds(NS * RPT, TAIL)])


def _acc_writeback(acc, out_hbm, tid):
    pltpu.sync_copy(acc.at[pl.ds(tid * RPT, RPT)],
                    out_hbm.at[pl.ds(tid * RPT, RPT)])

    @pl.when(tid == NS - 1)
    def _():
        pltpu.sync_copy(acc.at[pl.ds(NS * RPT, TAIL)],
                        out_hbm.at[pl.ds(NS * RPT, TAIL)])


def _ring_scatter(read_src_fn, didx_hbm, acc, rows0, rows1, didx0, didx1,
                  semg0, semg1, semi0, semi1, nch, base_t):
    def issue_read(i, rows, semg):
        pltpu.async_copy(read_src_fn(i), rows, semg)

    def wait_read(i, rows, semg):
        pltpu.make_async_copy(read_src_fn(i), rows, semg).wait()

    def issue_didx(i, didx, semi):
        pltpu.async_copy(didx_hbm.at[pl.ds(base_t + i * K, K)], didx, semi)

    def wait_didx(i, didx, semi):
        pltpu.make_async_copy(didx_hbm.at[pl.ds(base_t + i * K, K)], didx,
                              semi).wait()

    issue_didx(0, didx0, semi0)
    issue_read(0, rows0, semg0)

    def body(g, _):
        i0 = 2 * g
        i1 = i0 + 1
        issue_didx(i1, didx1, semi1)
        issue_read(i1, rows1, semg1)
        wait_read(i0, rows0, semg0)
        wait_didx(i0, didx0, semi0)
        pltpu.sync_copy(rows0, acc.at[didx0], add=True)

        @pl.when(i1 + 1 < nch)
        def _():
            issue_didx(i1 + 1, didx0, semi0)
            issue_read(i1 + 1, rows0, semg0)

        wait_read(i1, rows1, semg1)
        wait_didx(i1, didx1, semi1)
        pltpu.sync_copy(rows1, acc.at[didx1], add=True)
        return 0

    lax.fori_loop(0, nch // 2, body, 0)
    if nch % 2 == 1:
        i = nch - 1
        wait_read(i, rows0, semg0)
        wait_didx(i, didx0, semi0)
        pltpu.sync_copy(rows0, acc.at[didx0], add=True)


def _gather_diff_core(pos_hbm, neg_hbm, iv_hbm, iu_hbm, out_hbm, iall,
                      rows0, rows1, semg0, semg1, sema0, sema1, semw0, semw1,
                      tid, ept):
    nch = ept // K
    base_t = tid * ept
    pltpu.sync_copy(iv_hbm.at[pl.ds(base_t, ept)], iall.at[pl.ds(0, ept)])
    pltpu.sync_copy(iu_hbm.at[pl.ds(base_t, ept)], iall.at[pl.ds(ept, ept)])

    def g1_src(i):
        return pos_hbm.at[iall.at[pl.ds(i * K, K)]]

    def g2_src(i):
        return neg_hbm.at[iall.at[pl.ds(ept + i * K, K)]]

    def out_dst(i):
        return out_hbm.at[pl.ds(base_t + i * K, K)]

    pltpu.async_copy(g1_src(0), rows0, semg0)

    def body(g, _):
        i0 = 2 * g
        i1 = i0 + 1
        pltpu.make_async_copy(g1_src(i0), rows0, semg0).wait()
        pltpu.async_copy(g2_src(i0), rows0, sema0, add=True)

        @pl.when(g > 0)
        def _():
            pltpu.make_async_copy(rows1, out_dst(i1 - 2), semw1).wait()

        pltpu.async_copy(g1_src(i1), rows1, semg1)
        pltpu.make_async_copy(g2_src(i0), rows0, sema0).wait()
        pltpu.async_copy(rows0, out_dst(i0), semw0)
        pltpu.make_async_copy(g1_src(i1), rows1, semg1).wait()
        pltpu.async_copy(g2_src(i1), rows1, sema1, add=True)

        @pl.when(i1 + 1 < nch)
        def _():
            pltpu.make_async_copy(rows0, out_dst(i0), semw0).wait()
            pltpu.async_copy(g1_src(i1 + 1), rows0, semg0)

        pltpu.make_async_copy(g2_src(i1), rows1, sema1).wait()
        pltpu.async_copy(rows1, out_dst(i1), semw1)
        return 0

    lax.fori_loop(0, nch // 2, body, 0)
    i = nch - 1
    pltpu.make_async_copy(g1_src(i), rows0, semg0).wait()
    pltpu.async_copy(g2_src(i), rows0, sema0, add=True)
    pltpu.make_async_copy(g2_src(i), rows0, sema0).wait()
    pltpu.async_copy(rows0, out_dst(i), semw0)
    pltpu.make_async_copy(rows0, out_dst(i), semw0).wait()
    pltpu.make_async_copy(rows1, out_dst(i - 1), semw1).wait()


def _sc_stage_a(x, xw2, xw2n, adj_src, adj_dst, inc_v, inc_u):
    EA = adj_src.shape[0]
    EI = inc_v.shape[0]
    ept_a = EA // NS
    ept_i = EI // NS
    mesh = plsc.VectorSubcoreMesh(core_axis_name="c", subcore_axis_name="s")

    @functools.partial(
        pl.kernel,
        out_type=[jax.ShapeDtypeStruct((N, C), jnp.float32),
                  jax.ShapeDtypeStruct((EI, C), jnp.float32)],
        mesh=mesh,
        scratch_types=[
            pltpu.VMEM_SHARED((N, C), jnp.float32),
            pltpu.VMEM((2 * ept_i,), jnp.int32),
            pltpu.VMEM((K, C), jnp.float32),
            pltpu.VMEM((K, C), jnp.float32),
            pltpu.VMEM((K,), jnp.int32),
            pltpu.VMEM((K,), jnp.int32),
            pltpu.VMEM((ZR, C), jnp.float32),
            [pltpu.SemaphoreType.DMA] * 7,
        ],
    )
    def k(x_hbm, xw2_hbm, xw2n_hbm, asrc_hbm, adst_hbm, iv_hbm, iu_hbm,
          gx_hbm, epre_hbm, acc, iall, rows0, rows1, didx0, didx1, zbuf,
          sems):
        cid = lax.axis_index("c")
        tid = lax.axis_index("s")

        @pl.when(cid == 0)
        def _():
            nch = ept_a // K
            base_t = tid * ept_a
            _zero_acc_slice(acc, zbuf, tid, sems[6])
            pltpu.sync_copy(asrc_hbm.at[pl.ds(base_t, ept_a)],
                            iall.at[pl.ds(0, ept_a)])
            plsc.subcore_barrier()

            def read_src(i):
                return x_hbm.at[iall.at[pl.ds(i * K, K)]]

            _ring_scatter(read_src, adst_hbm, acc, rows0, rows1, didx0,
                          didx1, sems[0], sems[1], sems[2], sems[3], nch,
                          base_t)
            plsc.subcore_barrier()
            _acc_writeback(acc, gx_hbm, tid)

        @pl.when(cid == 1)
        def _():
            _gather_diff_core(xw2_hbm, xw2n_hbm, iv_hbm, iu_hbm, epre_hbm,
                              iall, rows0, rows1, sems[0], sems[1], sems[2],
                              sems[3], sems[4], sems[5], tid, ept_i)

    return k(x, xw2, xw2n, adj_src, adj_dst, inc_v, inc_u)


def _sc_stage_c(n1, ewp, ewn, adj_src, adj_dst, inc_v, inc_u):
    EA = adj_src.shape[0]
    EI = inc_v.shape[0]
    ept_a = EA // NS
    ept_i = EI // NS
    mesh = plsc.VectorSubcoreMesh(core_axis_name="c", subcore_axis_name="s")

    @functools.partial(
        pl.kernel,
        out_type=[jax.ShapeDtypeStruct((N, C), jnp.float32),
                  jax.ShapeDtypeStruct((N, C), jnp.float32)],
        mesh=mesh,
        scratch_types=[
            pltpu.VMEM_SHARED((N, C), jnp.float32),
            pltpu.VMEM((ept_a,), jnp.int32),
            pltpu.VMEM((K, C), jnp.float32),
            pltpu.VMEM((K, C), jnp.float32),
            pltpu.VMEM((K,), jnp.int32),
            pltpu.VMEM((K,), jnp.int32),
            pltpu.VMEM((ZR, C), jnp.float32),
            [pltpu.SemaphoreType.DMA] * 7,
        ],
    )
    def k(n1_hbm, ewp_hbm, ewn_hbm, asrc_hbm, adst_hbm, iv_hbm, iu_hbm,
          gn_hbm, e2_hbm, acc, iall, rows0, rows1, didx0, didx1, zbuf, sems):
        cid = lax.axis_index("c")
        tid = lax.axis_index("s")

        @pl.when(cid == 0)
        def _():
            nch = ept_a // K
            base_t = tid * ept_a
            _zero_acc_slice(acc, zbuf, tid, sems[6])
            pltpu.sync_copy(asrc_hbm.at[pl.ds(base_t, ept_a)],
                            iall.at[pl.ds(0, ept_a)])
            plsc.subcore_barrier()

            def read_src(i):
                return n1_hbm.at[iall.at[pl.ds(i * K, K)]]

            _ring_scatter(read_src, adst_hbm, acc, rows0, rows1, didx0,
                          didx1, sems[0], sems[1], sems[2], sems[3], nch,
                          base_t)
            plsc.subcore_barrier()
            _acc_writeback(acc, gn_hbm, tid)

        @pl.when(cid == 1)
        def _():
            nch = ept_i // K
            base_t = tid * ept_i
            _zero_acc_slice(acc, zbuf, tid, sems[6])
            plsc.subcore_barrier()

            def read_p(i):
                return ewp_hbm.at[pl.ds(base_t + i * K, K)]

            def read_n(i):
                return ewn_hbm.at[pl.ds(base_t + i * K, K)]

            _ring_scatter(read_p, iv_hbm, acc, rows0, rows1, didx0, didx1,
                          sems[0], sems[1], sems[2], sems[3], nch, base_t)
            _ring_scatter(read_n, iu_hbm, acc, rows0, rows1, didx0, didx1,
                          sems[0], sems[1], sems[2], sems[3], nch, base_t)
            plsc.subcore_barrier()
            _acc_writeback(acc, e2_hbm, tid)

    return k(n1, ewp, ewn, adj_src, adj_dst, inc_v, inc_u)


def kernel(x, adj_src, adj_dst, inc_u, inc_v, W1, W2, W3, W4):
    xw2, xw2n = _tc_pm(_mm_pm_body, x, W2, blk=1000)
    gx, e_pre = _sc_stage_a(x, xw2, xw2n, adj_src, adj_dst, inc_v, inc_u)
    n1 = _tc_mm_sig(gx, W1, blk=1000)
    ewp, ewn = _tc_pm(_sig_mm_pm_body, e_pre, W4, blk=2000)
    gn, e2 = _sc_stage_c(n1, ewp, ewn, adj_src, adj_dst, inc_v, inc_u)
    return _tc_mm_add_sig(gn, W3, e2, blk=1000)


# speedup vs baseline: 1.8432x; 1.1451x over previous
import functools

import jax
import jax.numpy as jnp
from jax import lax
from jax.experimental import pallas as pl
from jax.experimental.pallas import tpu as pltpu
from jax.experimental.pallas import tpu_sc as plsc

N = 10000
C = 128
NC = 2
NS = 16
ZR = 16
RPT = 624
TAIL = N - NS * RPT
K = 80


def _mm_pm_body(x_ref, w_ref, op_ref, on_ref):
    a = jnp.dot(x_ref[...], w_ref[...], preferred_element_type=jnp.float32)
    op_ref[...] = a
    on_ref[...] = -a


def _sig_mm_body(x_ref, w_ref, o_ref):
    s = jax.nn.sigmoid(x_ref[...])
    o_ref[...] = jnp.dot(s, w_ref[...], preferred_element_type=jnp.float32)


def _sum_mm_sig_body(a_ref, b_ref, w_ref, o_ref):
    o_ref[...] = jax.nn.sigmoid(
        jnp.dot(a_ref[...] + b_ref[...], w_ref[...],
                preferred_element_type=jnp.float32))


def _final_body(a_ref, b_ref, w_ref, p_ref, n_ref, o_ref):
    o_ref[...] = jax.nn.sigmoid(
        jnp.dot(a_ref[...] + b_ref[...], w_ref[...],
                preferred_element_type=jnp.float32)
        + p_ref[...] - n_ref[...])


def _row_spec(blk):
    return pl.BlockSpec((blk, C), lambda i: (i, 0))


def _w_spec():
    return pl.BlockSpec((C, C), lambda i: (0, 0))


def _tc_pm(x, w, blk):
    rows = x.shape[0]
    return pl.pallas_call(
        _mm_pm_body,
        grid=(rows // blk,),
        in_specs=[_row_spec(blk), _w_spec()],
        out_specs=[_row_spec(blk), _row_spec(blk)],
        out_shape=[jax.ShapeDtypeStruct((rows, C), jnp.float32)] * 2,
    )(x, w)


def _tc_sig_mm(x, w, blk):
    rows = x.shape[0]
    return pl.pallas_call(
        _sig_mm_body,
        grid=(rows // blk,),
        in_specs=[_row_spec(blk), _w_spec()],
        out_specs=_row_spec(blk),
        out_shape=jax.ShapeDtypeStruct((rows, C), jnp.float32),
    )(x, w)


def _tc_sum_mm_sig(a, b, w, blk):
    rows = a.shape[0]
    return pl.pallas_call(
        _sum_mm_sig_body,
        grid=(rows // blk,),
        in_specs=[_row_spec(blk), _row_spec(blk), _w_spec()],
        out_specs=_row_spec(blk),
        out_shape=jax.ShapeDtypeStruct((rows, C), jnp.float32),
    )(a, b, w)


def _tc_final(a, b, w, p, n, blk):
    rows = a.shape[0]
    return pl.pallas_call(
        _final_body,
        grid=(rows // blk,),
        in_specs=[_row_spec(blk), _row_spec(blk), _w_spec(),
                  _row_spec(blk), _row_spec(blk)],
        out_specs=_row_spec(blk),
        out_shape=jax.ShapeDtypeStruct((rows, C), jnp.float32),
    )(a, b, w, p, n)


def _zero_acc_slice(acc, zbuf, tid, sem):
    def zb(i, _):
        zbuf[i // (C // 16), pl.ds((i % (C // 16)) * 16, 16)] = (
            jnp.zeros((16,), jnp.float32))
        return 0

    lax.fori_loop(0, ZR * (C // 16), zb, 0)

    def zcopy(i, _):
        pltpu.async_copy(zbuf, acc.at[pl.ds(tid * RPT + i * ZR, ZR)], sem)
        return 0

    lax.fori_loop(0, RPT // ZR, zcopy, 0)

    def zdrain(i, _):
        pltpu.make_async_copy(zbuf, acc.at[pl.ds(tid * RPT + i * ZR, ZR)],
                              sem).wait()
        return 0

    lax.fori_loop(0, RPT // ZR, zdrain, 0)

    @pl.when(tid == NS - 1)
    def _():
        pltpu.sync_copy(zbuf.at[pl.ds(0, TAIL)], acc.at[pl.ds(NS * RPT, TAIL)])


def _acc_writeback(acc, out_hbm, tid):
    pltpu.sync_copy(acc.at[pl.ds(tid * RPT, RPT)],
                    out_hbm.at[pl.ds(tid * RPT, RPT)])

    @pl.when(tid == NS - 1)
    def _():
        pltpu.sync_copy(acc.at[pl.ds(NS * RPT, TAIL)],
                        out_hbm.at[pl.ds(NS * RPT, TAIL)])


def _ring_scatter(read_src_fn, didx_hbm, acc, rows0, rows1, didx0, didx1,
                  semg0, semg1, semi0, semi1, nch, base_t):
    def issue_read(i, rows, semg):
        pltpu.async_copy(read_src_fn(i), rows, semg)

    def wait_read(i, rows, semg):
        pltpu.make_async_copy(read_src_fn(i), rows, semg).wait()

    def issue_didx(i, didx, semi):
        pltpu.async_copy(didx_hbm.at[pl.ds(base_t + i * K, K)], didx, semi)

    def wait_didx(i, didx, semi):
        pltpu.make_async_copy(didx_hbm.at[pl.ds(base_t + i * K, K)], didx,
                              semi).wait()

    issue_didx(0, didx0, semi0)
    issue_read(0, rows0, semg0)

    def body(g, _):
        i0 = 2 * g
        i1 = i0 + 1
        issue_didx(i1, didx1, semi1)
        issue_read(i1, rows1, semg1)
        wait_read(i0, rows0, semg0)
        wait_didx(i0, didx0, semi0)
        pltpu.sync_copy(rows0, acc.at[didx0], add=True)

        @pl.when(i1 + 1 < nch)
        def _():
            issue_didx(i1 + 1, didx0, semi0)
            issue_read(i1 + 1, rows0, semg0)

        wait_read(i1, rows1, semg1)
        wait_didx(i1, didx1, semi1)
        pltpu.sync_copy(rows1, acc.at[didx1], add=True)
        return 0

    lax.fori_loop(0, nch // 2, body, 0)
    if nch % 2 == 1:
        i = nch - 1
        wait_read(i, rows0, semg0)
        wait_didx(i, didx0, semi0)
        pltpu.sync_copy(rows0, acc.at[didx0], add=True)


def _gather_diff_range(pos_hbm, neg_hbm, iv_hbm, iu_hbm, out_hbm, iall,
                       rows0, rows1, semg0, semg1, sema0, sema1, semw0,
                       semw1, base_t, ept):
    nch = ept // K
    pltpu.sync_copy(iv_hbm.at[pl.ds(base_t, ept)], iall.at[pl.ds(0, ept)])
    pltpu.sync_copy(iu_hbm.at[pl.ds(base_t, ept)], iall.at[pl.ds(ept, ept)])

    def g1_src(i):
        return pos_hbm.at[iall.at[pl.ds(i * K, K)]]

    def g2_src(i):
        return neg_hbm.at[iall.at[pl.ds(ept + i * K, K)]]

    def out_dst(i):
        return out_hbm.at[pl.ds(base_t + i * K, K)]

    pltpu.async_copy(g1_src(0), rows0, semg0)

    def body(g, _):
        i0 = 2 * g
        i1 = i0 + 1
        pltpu.make_async_copy(g1_src(i0), rows0, semg0).wait()
        pltpu.async_copy(g2_src(i0), rows0, sema0, add=True)

        @pl.when(g > 0)
        def _():
            pltpu.make_async_copy(rows1, out_dst(i1 - 2), semw1).wait()

        pltpu.async_copy(g1_src(i1), rows1, semg1)
        pltpu.make_async_copy(g2_src(i0), rows0, sema0).wait()
        pltpu.async_copy(rows0, out_dst(i0), semw0)
        pltpu.make_async_copy(g1_src(i1), rows1, semg1).wait()
        pltpu.async_copy(g2_src(i1), rows1, sema1, add=True)

        @pl.when(i1 + 1 < nch)
        def _():
            pltpu.make_async_copy(rows0, out_dst(i0), semw0).wait()
            pltpu.async_copy(g1_src(i1 + 1), rows0, semg0)

        pltpu.make_async_copy(g2_src(i1), rows1, sema1).wait()
        pltpu.async_copy(rows1, out_dst(i1), semw1)
        return 0

    lax.fori_loop(0, nch // 2, body, 0)
    if nch % 2 == 1:
        i = nch - 1
        pltpu.make_async_copy(g1_src(i), rows0, semg0).wait()
        pltpu.async_copy(g2_src(i), rows0, sema0, add=True)
        pltpu.make_async_copy(g2_src(i), rows0, sema0).wait()
        pltpu.async_copy(rows0, out_dst(i), semw0)
        pltpu.make_async_copy(rows0, out_dst(i), semw0).wait()
        pltpu.make_async_copy(rows1, out_dst(i - 1), semw1).wait()
    else:
        pltpu.make_async_copy(rows0, out_dst(nch - 2), semw0).wait()
        pltpu.make_async_copy(rows1, out_dst(nch - 1), semw1).wait()


def _sc_adj(table, adj_src, adj_dst):
    EA = adj_src.shape[0]
    ept = EA // (NC * NS)
    mesh = plsc.VectorSubcoreMesh(core_axis_name="c", subcore_axis_name="s")

    @functools.partial(
        pl.kernel,
        out_type=[jax.ShapeDtypeStruct((N, C), jnp.float32),
                  jax.ShapeDtypeStruct((N, C), jnp.float32)],
        mesh=mesh,
        scratch_types=[
            pltpu.VMEM_SHARED((N, C), jnp.float32),
            pltpu.VMEM((ept,), jnp.int32),
            pltpu.VMEM((K, C), jnp.float32),
            pltpu.VMEM((K, C), jnp.float32),
            pltpu.VMEM((K,), jnp.int32),
            pltpu.VMEM((K,), jnp.int32),
            pltpu.VMEM((ZR, C), jnp.float32),
            [pltpu.SemaphoreType.DMA] * 5,
        ],
    )
    def k(t_hbm, asrc_hbm, adst_hbm, p0_hbm, p1_hbm, acc, iall, rows0,
          rows1, didx0, didx1, zbuf, sems):
        cid = lax.axis_index("c")
        tid = lax.axis_index("s")
        nch = ept // K
        base_t = (cid * NS + tid) * ept
        _zero_acc_slice(acc, zbuf, tid, sems[4])
        pltpu.sync_copy(asrc_hbm.at[pl.ds(base_t, ept)],
                        iall.at[pl.ds(0, ept)])
        plsc.subcore_barrier()

        def read_src(i):
            return t_hbm.at[iall.at[pl.ds(i * K, K)]]

        _ring_scatter(read_src, adst_hbm, acc, rows0, rows1, didx0, didx1,
                      sems[0], sems[1], sems[2], sems[3], nch, base_t)
        plsc.subcore_barrier()

        @pl.when(cid == 0)
        def _():
            _acc_writeback(acc, p0_hbm, tid)

        @pl.when(cid == 1)
        def _():
            _acc_writeback(acc, p1_hbm, tid)

    return k(table, adj_src, adj_dst)


def _sc_gdiff(xw2, xw2n, inc_v, inc_u):
    EI = inc_v.shape[0]
    e0 = 5040
    e1 = 4960
    split = NS * e0
    assert NS * (e0 + e1) == EI
    mesh = plsc.VectorSubcoreMesh(core_axis_name="c", subcore_axis_name="s")

    @functools.partial(
        pl.kernel,
        out_type=jax.ShapeDtypeStruct((EI, C), jnp.float32),
        mesh=mesh,
        scratch_types=[
            pltpu.VMEM((2 * e0,), jnp.int32),
            pltpu.VMEM((K, C), jnp.float32),
            pltpu.VMEM((K, C), jnp.float32),
            [pltpu.SemaphoreType.DMA] * 6,
        ],
    )
    def k(xw2_hbm, xw2n_hbm, iv_hbm, iu_hbm, epre_hbm, iall, rows0, rows1,
          sems):
        cid = lax.axis_index("c")
        tid = lax.axis_index("s")

        @pl.when(cid == 0)
        def _():
            _gather_diff_range(xw2_hbm, xw2n_hbm, iv_hbm, iu_hbm, epre_hbm,
                               iall, rows0, rows1, sems[0], sems[1], sems[2],
                               sems[3], sems[4], sems[5], tid * e0, e0)

        @pl.when(cid == 1)
        def _():
            _gather_diff_range(xw2_hbm, xw2n_hbm, iv_hbm, iu_hbm, epre_hbm,
                               iall, rows0, rows1, sems[0], sems[1], sems[2],
                               sems[3], sems[4], sems[5], split + tid * e1,
                               e1)

    return k(xw2, xw2n, inc_v, inc_u)


def _sc_inc_scatter(ew, inc_v, inc_u):
    EI = inc_v.shape[0]
    ept = EI // NS
    mesh = plsc.VectorSubcoreMesh(core_axis_name="c", subcore_axis_name="s")

    @functools.partial(
        pl.kernel,
        out_type=[jax.ShapeDtypeStruct((N, C), jnp.float32),
                  jax.ShapeDtypeStruct((N, C), jnp.float32)],
        mesh=mesh,
        scratch_types=[
            pltpu.VMEM_SHARED((N, C), jnp.float32),
            pltpu.VMEM((K, C), jnp.float32),
            pltpu.VMEM((K, C), jnp.float32),
            pltpu.VMEM((K,), jnp.int32),
            pltpu.VMEM((K,), jnp.int32),
            pltpu.VMEM((ZR, C), jnp.float32),
            [pltpu.SemaphoreType.DMA] * 5,
        ],
    )
    def k(ew_hbm, iv_hbm, iu_hbm, e2p_hbm, e2n_hbm, acc, rows0, rows1,
          didx0, didx1, zbuf, sems):
        cid = lax.axis_index("c")
        tid = lax.axis_index("s")
        nch = ept // K
        base_t = tid * ept
        _zero_acc_slice(acc, zbuf, tid, sems[4])
        plsc.subcore_barrier()

        def read_src(i):
            return ew_hbm.at[pl.ds(base_t + i * K, K)]

        @pl.when(cid == 0)
        def _():
            _ring_scatter(read_src, iv_hbm, acc, rows0, rows1, didx0, didx1,
                          sems[0], sems[1], sems[2], sems[3], nch, base_t)

        @pl.when(cid == 1)
        def _():
            _ring_scatter(read_src, iu_hbm, acc, rows0, rows1, didx0, didx1,
                          sems[0], sems[1], sems[2], sems[3], nch, base_t)

        plsc.subcore_barrier()

        @pl.when(cid == 0)
        def _():
            _acc_writeback(acc, e2p_hbm, tid)

        @pl.when(cid == 1)
        def _():
            _acc_writeback(acc, e2n_hbm, tid)

    return k(ew, inc_v, inc_u)


def kernel(x, adj_src, adj_dst, inc_u, inc_v, W1, W2, W3, W4):
    xw2, xw2n = _tc_pm(x, W2, blk=1000)
    gxp0, gxp1 = _sc_adj(x, adj_src, adj_dst)
    e_pre = _sc_gdiff(xw2, xw2n, inc_v, inc_u)
    n1 = _tc_sum_mm_sig(gxp0, gxp1, W1, blk=1000)
    ew = _tc_sig_mm(e_pre, W4, blk=2000)
    gnp0, gnp1 = _sc_adj(n1, adj_src, adj_dst)
    e2p, e2n = _sc_inc_scatter(ew, inc_v, inc_u)
    return _tc_final(gnp0, gnp1, W3, e2p, e2n, blk=1000)
